# baseline (speedup 1.0000x reference)
import functools
import math

import jax
import jax.numpy as jnp
from jax import lax
from jax.experimental import pallas as pl
from jax.experimental.pallas import tpu as pltpu
from jax.experimental.pallas import tpu_sc as plsc

D_MODEL = 128
MAX_SEQ_LEN = 200
BATCH = 1024
_SCALE = math.sqrt(float(D_MODEL))

NC = 2
NS = 16
NW = NC * NS
ROWS_PER_W = BATCH // NW
HALF = MAX_SEQ_LEN // 2
LANES = 16
NVEC = D_MODEL // LANES


def _pos_encoding():
    even_i = jnp.arange(0, D_MODEL, 2, dtype=jnp.float32)
    denominator = jnp.power(even_i, even_i / D_MODEL)
    position = jnp.arange(MAX_SEQ_LEN, dtype=jnp.float32).reshape(MAX_SEQ_LEN, 1)
    even_pe = jnp.sin(position / denominator)
    odd_pe = jnp.cos(position / denominator)
    stacked = jnp.stack([even_pe, odd_pe], axis=-1)
    return stacked.reshape(MAX_SEQ_LEN, D_MODEL)


def _embed_encode(x3, pe, table):
    mesh = plsc.VectorSubcoreMesh(core_axis_name="c", subcore_axis_name="s")

    @functools.partial(
        pl.kernel,
        out_type=jax.ShapeDtypeStruct((BATCH, MAX_SEQ_LEN, D_MODEL), jnp.float32),
        mesh=mesh,
        scratch_types=[
            pltpu.VMEM((MAX_SEQ_LEN, D_MODEL), jnp.float32),
            pltpu.VMEM((2, HALF), jnp.int32),
            pltpu.VMEM((MAX_SEQ_LEN, D_MODEL), jnp.float32),
            pltpu.SemaphoreType.DMA,
        ],
    )
    def k(x_hbm, pe_hbm, table_hbm, out_hbm, pe_v, idx_v, rows_v, sem):
        wid = lax.axis_index("s") * NC + lax.axis_index("c")
        pltpu.sync_copy(pe_hbm, pe_v)

        def body(i, _):
            b = wid * ROWS_PER_W + i
            pltpu.sync_copy(x_hbm.at[b], idx_v)
            cp0 = pltpu.async_copy(
                table_hbm.at[idx_v.at[0]], rows_v.at[pl.ds(0, HALF)], sem)
            cp1 = pltpu.async_copy(
                table_hbm.at[idx_v.at[1]], rows_v.at[pl.ds(HALF, HALF)], sem)
            cp0.wait()
            cp1.wait()

            def crow(r, _):
                for c in range(NVEC):
                    sl = pl.ds(c * LANES, LANES)
                    rows_v[r, sl] = rows_v[r, sl] * _SCALE + pe_v[r, sl]
                return ()

            lax.fori_loop(0, MAX_SEQ_LEN, crow, ())
            pltpu.sync_copy(rows_v, out_hbm.at[b])
            return ()

        lax.fori_loop(0, ROWS_PER_W, body, ())

    return k(x3, pe, table)


def kernel(x, table):
    x3 = x.reshape(BATCH, 2, HALF)
    pe = _pos_encoding()
    return _embed_encode(x3, pe, table)


# speedup vs baseline: 1.5393x; 1.5393x over previous
import functools
import math

import jax
import jax.numpy as jnp
from jax import lax
from jax.experimental import pallas as pl
from jax.experimental.pallas import tpu as pltpu
from jax.experimental.pallas import tpu_sc as plsc

D_MODEL = 128
MAX_SEQ_LEN = 200
BATCH = 1024
_SCALE = math.sqrt(float(D_MODEL))

NC = 2
NS = 16
NW = NC * NS
ROWS_PER_W = BATCH // NW
HALF = MAX_SEQ_LEN // 2
LANES = 16
NVEC = D_MODEL // LANES
NBUF = 3


def _pos_encoding():
    even_i = jnp.arange(0, D_MODEL, 2, dtype=jnp.float32)
    denominator = jnp.power(even_i, even_i / D_MODEL)
    position = jnp.arange(MAX_SEQ_LEN, dtype=jnp.float32).reshape(MAX_SEQ_LEN, 1)
    even_pe = jnp.sin(position / denominator)
    odd_pe = jnp.cos(position / denominator)
    stacked = jnp.stack([even_pe, odd_pe], axis=-1)
    return stacked.reshape(MAX_SEQ_LEN, D_MODEL)


def _embed_encode(x3, pe, table):
    mesh = plsc.VectorSubcoreMesh(core_axis_name="c", subcore_axis_name="s")

    scratch = [pltpu.VMEM((MAX_SEQ_LEN, D_MODEL), jnp.float32)]
    scratch += [pltpu.VMEM((2, HALF), jnp.int32) for _ in range(NBUF)]
    scratch += [pltpu.VMEM((MAX_SEQ_LEN, D_MODEL), jnp.float32)
                for _ in range(NBUF)]
    scratch += [pltpu.SemaphoreType.DMA for _ in range(2 * NBUF)]

    @functools.partial(
        pl.kernel,
        out_type=jax.ShapeDtypeStruct((BATCH, MAX_SEQ_LEN, D_MODEL), jnp.float32),
        mesh=mesh,
        scratch_types=scratch,
    )
    def k(x_hbm, pe_hbm, table_hbm, out_hbm, pe_v, *rest):
        idxs = rest[:NBUF]
        rows = rest[NBUF:2 * NBUF]
        gsems = rest[2 * NBUF:3 * NBUF]
        osems = rest[3 * NBUF:4 * NBUF]

        wid = lax.axis_index("s") * NC + lax.axis_index("c")
        base = wid * ROWS_PER_W
        pltpu.sync_copy(pe_hbm, pe_v)

        def issue_gather(i, p):
            pltpu.sync_copy(x_hbm.at[base + i], idxs[p])
            pltpu.async_copy(table_hbm.at[idxs[p].at[0]],
                             rows[p].at[pl.ds(0, HALF)], gsems[p])
            pltpu.async_copy(table_hbm.at[idxs[p].at[1]],
                             rows[p].at[pl.ds(HALF, HALF)], gsems[p])

        def wait_gather(p):
            pltpu.make_async_copy(table_hbm.at[idxs[p].at[0]],
                                  rows[p].at[pl.ds(0, HALF)], gsems[p]).wait()
            pltpu.make_async_copy(table_hbm.at[idxs[p].at[1]],
                                  rows[p].at[pl.ds(HALF, HALF)], gsems[p]).wait()

        def issue_out(i, p):
            pltpu.async_copy(rows[p], out_hbm.at[base + i], osems[p])

        def wait_out(i, p):
            pltpu.make_async_copy(rows[p], out_hbm.at[base + i], osems[p]).wait()

        def compute(p):
            rowsb = rows[p]

            @plsc.parallel_loop(0, MAX_SEQ_LEN, step=2)
            def _(r):
                for rr in range(2):
                    for c in range(NVEC):
                        sl = pl.ds(c * LANES, LANES)
                        rowsb[r + rr, sl] = (
                            rowsb[r + rr, sl] * _SCALE + pe_v[r + rr, sl])

        issue_gather(0, 0)
        for i in range(ROWS_PER_W):
            p = i % NBUF
            nxt = i + 1
            if nxt < ROWS_PER_W:
                pn = nxt % NBUF
                if nxt >= NBUF:
                    wait_out(nxt - NBUF, pn)
                issue_gather(nxt, pn)
            wait_gather(p)
            compute(p)
            issue_out(i, p)
        for i in range(ROWS_PER_W - NBUF, ROWS_PER_W):
            wait_out(i, i % NBUF)

    return k(x3, pe, table)


def kernel(x, table):
    x3 = x.reshape(BATCH, 2, HALF)
    pe = _pos_encoding()
    return _embed_encode(x3, pe, table)
